# Initial kernel scaffold; baseline (speedup 1.0000x reference)
import jax
import jax.numpy as jnp
from jax.experimental import pallas as pl


def kernel(x, edge_index, fi, w1, b1, w2, b2, we, be, wh0, bh0, wh1, bh1, bn_gamma, bn_beta, wc, bc, gate_temp):
    raise NotImplementedError("write your pallas kernel here")


# speedup vs baseline: 5.8300x; 5.8300x over previous
import functools

import jax
import jax.numpy as jnp
from jax import lax
from jax.experimental import pallas as pl
from jax.experimental.pallas import tpu as pltpu
from jax.experimental.pallas import tpu_sc as plsc

N = 10000
F = 128
E = 320000
HID = 128
OUT = 2
HOP = 42
NC = 2
NS = 16
N_PAD = 10240
DUMMY = N_PAD - 1
CHUNK = 128
E_PAD = 323584
RPT = N_PAD // NS
BLK = 512

_mesh = plsc.VectorSubcoreMesh(
    core_axis_name="c", subcore_axis_name="s", num_cores=NC, num_subcores=NS)


def _zero_acc(znf, acc, rbase):
    pltpu.sync_copy(znf.at[pl.ds(rbase, RPT)], acc.at[pl.ds(rbase, RPT)])


def _sc0_body(srcx, dstx, znf, onef,
              h_out,
              idx_s, ones_v, hacc):
    cid = lax.axis_index("c")
    sid = lax.axis_index("s")
    rbase = sid * RPT
    _zero_acc(znf, hacc, rbase)
    pltpu.sync_copy(onef, ones_v)
    plsc.subcore_barrier()
    per_t = E_PAD // NS
    ebase = sid * per_t

    def run(sidx):
        def body(c, carry):
            b = ebase + c * CHUNK
            pltpu.sync_copy(sidx.at[pl.ds(b, CHUNK)], idx_s)
            pltpu.sync_copy(ones_v, hacc.at[idx_s], add=True)
            return carry
        lax.fori_loop(0, per_t // CHUNK, body, 0)

    @pl.when(cid == 0)
    def _():
        run(srcx)

    @pl.when(cid == 1)
    def _():
        run(dstx)

    plsc.subcore_barrier()
    pltpu.sync_copy(hacc.at[pl.ds(rbase, RPT)], h_out.at[cid, pl.ds(rbase, RPT)])


def _sc1_body(table, srcx, dstx, znf,
              s1_out,
              idx_g, idx_s, rows, acc, sem):
    cid = lax.axis_index("c")
    sid = lax.axis_index("s")
    wid = sid * NC + cid
    rbase = sid * RPT
    _zero_acc(znf, acc, rbase)
    plsc.subcore_barrier()
    per_w = E_PAD // (NC * NS)
    ebase = wid * per_w

    def body(c, carry):
        b = ebase + c * CHUNK
        pltpu.sync_copy(dstx.at[pl.ds(b, CHUNK)], idx_g)
        pltpu.sync_copy(srcx.at[pl.ds(b, CHUNK)], idx_s)
        pltpu.async_copy(table.at[idx_g], rows, sem).wait()
        pltpu.sync_copy(rows, acc.at[idx_s], add=True)
        return carry

    lax.fori_loop(0, per_w // CHUNK, body, 0)
    plsc.subcore_barrier()
    pltpu.sync_copy(acc.at[pl.ds(rbase, RPT)], s1_out.at[cid, pl.ds(rbase, RPT)])


def _sc2_body(t_a, t_b, srcx, dstx, znf,
              out2,
              idx_g, idx_s, rows, acc, sem):
    cid = lax.axis_index("c")
    sid = lax.axis_index("s")
    rbase = sid * RPT
    _zero_acc(znf, acc, rbase)
    plsc.subcore_barrier()
    per_t = E_PAD // NS
    ebase = sid * per_t

    def run(table, gidx, sidx):
        def body(c, carry):
            b = ebase + c * CHUNK
            pltpu.sync_copy(gidx.at[pl.ds(b, CHUNK)], idx_g)
            pltpu.sync_copy(sidx.at[pl.ds(b, CHUNK)], idx_s)
            pltpu.async_copy(table.at[idx_g], rows, sem).wait()
            pltpu.sync_copy(rows, acc.at[idx_s], add=True)
            return carry
        lax.fori_loop(0, per_t // CHUNK, body, 0)

    @pl.when(cid == 0)
    def _():
        run(t_a, dstx, srcx)

    @pl.when(cid == 1)
    def _():
        run(t_b, srcx, dstx)

    plsc.subcore_barrier()
    pltpu.sync_copy(acc.at[pl.ds(rbase, RPT)], out2.at[cid, pl.ds(rbase, RPT)])


def _sc3_body(table, srcx, dstx, znf,
              out2,
              idx_g, idx_s, rows, acc, sem):
    cid = lax.axis_index("c")
    sid = lax.axis_index("s")
    wid = sid * NC + cid
    rbase = sid * RPT
    _zero_acc(znf, acc, rbase)
    plsc.subcore_barrier()
    per_w = E_PAD // (NC * NS)
    ebase = wid * per_w

    def body(c, carry):
        b = ebase + c * CHUNK
        pltpu.sync_copy(srcx.at[pl.ds(b, CHUNK)], idx_g)
        pltpu.sync_copy(dstx.at[pl.ds(b, CHUNK)], idx_s)
        pltpu.async_copy(table.at[idx_g], rows, sem).wait()
        pltpu.sync_copy(rows, acc.at[idx_s], add=True)
        return carry

    lax.fori_loop(0, per_w // CHUNK, body, 0)
    plsc.subcore_barrier()
    pltpu.sync_copy(acc.at[pl.ds(rbase, RPT)], out2.at[cid, pl.ds(rbase, RPT)])


_f32 = jnp.float32

_sc0 = functools.partial(
    pl.kernel, _sc0_body,
    out_type=jax.ShapeDtypeStruct((NC, N_PAD, F), _f32),
    mesh=_mesh,
    scratch_types=[pltpu.VMEM((CHUNK,), jnp.int32),
                   pltpu.VMEM((CHUNK, F), _f32),
                   pltpu.VMEM_SHARED((N_PAD, F), _f32)],
)()

_sc1 = functools.partial(
    pl.kernel, _sc1_body,
    out_type=jax.ShapeDtypeStruct((NC, N_PAD, F), _f32),
    mesh=_mesh,
    scratch_types=[pltpu.VMEM((CHUNK,), jnp.int32),
                   pltpu.VMEM((CHUNK,), jnp.int32),
                   pltpu.VMEM((CHUNK, F), _f32),
                   pltpu.VMEM_SHARED((N_PAD, F), _f32),
                   pltpu.SemaphoreType.DMA],
)()

_sc2 = functools.partial(
    pl.kernel, _sc2_body,
    out_type=jax.ShapeDtypeStruct((NC, N_PAD, F), _f32),
    mesh=_mesh,
    scratch_types=[pltpu.VMEM((CHUNK,), jnp.int32),
                   pltpu.VMEM((CHUNK,), jnp.int32),
                   pltpu.VMEM((CHUNK, F), _f32),
                   pltpu.VMEM_SHARED((N_PAD, F), _f32),
                   pltpu.SemaphoreType.DMA],
)()

_sc3 = functools.partial(
    pl.kernel, _sc3_body,
    out_type=jax.ShapeDtypeStruct((NC, N_PAD, F), _f32),
    mesh=_mesh,
    scratch_types=[pltpu.VMEM((CHUNK,), jnp.int32),
                   pltpu.VMEM((CHUNK,), jnp.int32),
                   pltpu.VMEM((CHUNK, F), _f32),
                   pltpu.VMEM_SHARED((N_PAD, F), _f32),
                   pltpu.SemaphoreType.DMA],
)()


def _tca_body(x_ref, fi_ref, w1_ref, xw_ref, xw1_ref):
    xw = x_ref[...] * jax.nn.sigmoid(fi_ref[...])
    xw_ref[...] = xw
    xw1_ref[...] = jnp.dot(xw, w1_ref[...], preferred_element_type=_f32)


def _tcb_body(s1_ref, h_ref, xw_ref, xw1_ref, gt_ref,
              hhop_ref, p1_ref, aux_ref):
    s1 = s1_ref[0] + s1_ref[1]
    deg_s = h_ref[0][:, 0:1]
    deg_d = h_ref[1][:, 0:1]
    dsinv = 1.0 / jnp.maximum(deg_s, 1.0)
    mean_nb = s1 * dsinv
    xw = xw_ref[...]
    nx = jnp.sqrt(jnp.sum(xw * xw, axis=1, keepdims=True))
    nm = jnp.sqrt(jnp.sum(mean_nb * mean_nb, axis=1, keepdims=True))
    dot = jnp.sum(xw * mean_nb, axis=1, keepdims=True)
    sim = dot / (jnp.maximum(nx, 1e-12) * jnp.maximum(nm, 1e-12))
    sim = jnp.where(deg_s > 0, sim, 1.0)
    delta = jax.nn.sigmoid(deg_s * (1.0 - sim) / 10.0 - 0.5)
    gate = jax.nn.sigmoid((delta - 0.5) * gt_ref[0, 0] * 10.0)
    dinv = lax.rsqrt(deg_d + 1.0)
    dinv2 = dinv * dinv
    hhop_ref[...] = mean_nb
    p1_ref[...] = dinv * xw1_ref[...]
    lane = lax.broadcasted_iota(jnp.int32, s1.shape, 1)
    aux = jnp.where(lane == 0, gate,
                    jnp.where(lane == 1, dinv,
                              jnp.where(lane == 2, dinv2, dsinv)))
    aux_ref[...] = aux


def _tcc_body(c1_ref, xw1_ref, aux_ref, w2_ref, b1_ref, p2_ref, g1_ref):
    dinv = aux_ref[:, 1:2]
    dinv2 = aux_ref[:, 2:3]
    h1 = jax.nn.relu(dinv * c1_ref[0] + dinv2 * xw1_ref[...] + b1_ref[...])
    g1 = jnp.dot(h1, w2_ref[...], preferred_element_type=_f32)
    g1_ref[...] = g1
    p2_ref[...] = dinv * g1


def _tcd_body(c2_ref, g1_ref, hhop_ref, xw_ref, s2_ref, aux_ref,
              wep_ref, wh0_ref, wh1_ref, bhigh_ref, b2_ref,
              bng_ref, bnb_ref, wcp_ref, bcp_ref, out_ref):
    gate = aux_ref[:, 0:1]
    dinv = aux_ref[:, 1:2]
    dinv2 = aux_ref[:, 2:3]
    dsinv = aux_ref[:, 3:4]
    h_low = dinv * (c2_ref[0] + c2_ref[1]) + dinv2 * g1_ref[...] + b2_ref[...]
    hop2 = s2_ref[0] * dsinv
    h_high = jax.nn.relu(
        jnp.dot(xw_ref[...], wep_ref[...], preferred_element_type=_f32)
        + jnp.dot(hhop_ref[...], wh0_ref[...], preferred_element_type=_f32)
        + jnp.dot(hop2, wh1_ref[...], preferred_element_type=_f32)
        + bhigh_ref[...])
    h = (1.0 - gate) * h_low + gate * h_high
    h = (h / jnp.sqrt(1.0 + 1e-5)) * bng_ref[...] + bnb_ref[...]
    out_ref[...] = jnp.dot(h, wcp_ref[...], preferred_element_type=_f32) + bcp_ref[...]


_GRID = (N_PAD // BLK,)


def _row_spec(w=F):
    return pl.BlockSpec((BLK, w), lambda i: (i, 0))


def _full_spec(shape):
    nd = len(shape)
    return pl.BlockSpec(shape, lambda i: (0,) * nd)


def _part_spec(w=F):
    return pl.BlockSpec((1, BLK, w), lambda i: (0, i, 0))


def _part2_spec(w=F):
    return pl.BlockSpec((2, BLK, w), lambda i: (0, i, 0))


def _slot_spec(slot, w=F):
    return pl.BlockSpec((1, BLK, w), lambda i: (slot, i, 0))


_tca = pl.pallas_call(
    _tca_body, grid=_GRID,
    in_specs=[_row_spec(), _full_spec((1, F)), _full_spec((F, HID))],
    out_specs=[_row_spec(), _row_spec()],
    out_shape=[jax.ShapeDtypeStruct((N_PAD, F), _f32),
               jax.ShapeDtypeStruct((N_PAD, HID), _f32)],
)

_tcb = pl.pallas_call(
    _tcb_body, grid=_GRID,
    in_specs=[_part2_spec(), _part2_spec(),
              _row_spec(), _row_spec(), _full_spec((1, 1))],
    out_specs=[_row_spec(), _row_spec(), _row_spec()],
    out_shape=[jax.ShapeDtypeStruct((N_PAD, F), _f32),
               jax.ShapeDtypeStruct((N_PAD, HID), _f32),
               jax.ShapeDtypeStruct((N_PAD, F), _f32)],
)

_tcc = pl.pallas_call(
    _tcc_body, grid=_GRID,
    in_specs=[_slot_spec(1), _row_spec(), _row_spec(),
              _full_spec((HID, HID)), _full_spec((1, HID))],
    out_specs=[_row_spec(), _row_spec()],
    out_shape=[jax.ShapeDtypeStruct((N_PAD, HID), _f32),
               jax.ShapeDtypeStruct((N_PAD, HID), _f32)],
)

_tcd = pl.pallas_call(
    _tcd_body, grid=_GRID,
    in_specs=[_part2_spec(), _row_spec(), _row_spec(), _row_spec(),
              _slot_spec(0), _row_spec(),
              _full_spec((F, HID)), _full_spec((F, HID)), _full_spec((F, HID)),
              _full_spec((1, HID)), _full_spec((1, HID)),
              _full_spec((1, HID)), _full_spec((1, HID)),
              _full_spec((HID, HID)), _full_spec((1, HID))],
    out_specs=_row_spec(),
    out_shape=jax.ShapeDtypeStruct((N_PAD, HID), _f32),
)


def kernel(x, edge_index, fi, w1, b1, w2, b2, we, be, wh0, bh0, wh1, bh1,
           bn_gamma, bn_beta, wc, bc, gate_temp):
    src = edge_index[0]
    dst = edge_index[1]
    src_p = jnp.full((E_PAD,), DUMMY, jnp.int32).at[:E].set(src)
    dst_p = jnp.full((E_PAD,), DUMMY, jnp.int32).at[:E].set(dst)
    x_p = jnp.zeros((N_PAD, F), _f32).at[:N].set(x)
    znf = jnp.zeros((N_PAD, F), _f32)
    onef = jnp.ones((CHUNK, F), _f32)

    wep = jnp.zeros((F, HID), _f32).at[:, :HOP].set(we)
    wh0p = jnp.zeros((F, HID), _f32).at[:, HOP:2 * HOP].set(wh0)
    wh1p = jnp.zeros((F, HID), _f32).at[:, 2 * HOP:].set(wh1)
    bhigh = jnp.concatenate([be, bh0, bh1]).reshape(1, HID)
    wcp = jnp.zeros((HID, HID), _f32).at[:, :OUT].set(wc)
    bcp = jnp.zeros((HID,), _f32).at[:OUT].set(bc).reshape(1, HID)

    hist = _sc0(src_p, dst_p, znf, onef)
    xw, xw1 = _tca(x_p, fi.reshape(1, F), w1)
    s1p = _sc1(xw, src_p, dst_p, znf)
    hhop, p1, aux = _tcb(s1p, hist, xw, xw1,
                         gate_temp.reshape(1, 1).astype(_f32))
    out2 = _sc2(hhop, p1, src_p, dst_p, znf)
    p2, g1 = _tcc(out2, xw1, aux, w2, b1.reshape(1, HID))
    c2p = _sc3(p2, src_p, dst_p, znf)
    final = _tcd(c2p, g1, hhop, xw, out2, aux, wep, wh0p, wh1p, bhigh,
                 b2.reshape(1, HID), bn_gamma.reshape(1, HID),
                 bn_beta.reshape(1, HID), wcp, bcp)
    return final[:N, :OUT]


# speedup vs baseline: 7.8993x; 1.3549x over previous
import functools

import jax
import jax.numpy as jnp
from jax import lax
from jax.experimental import pallas as pl
from jax.experimental.pallas import tpu as pltpu
from jax.experimental.pallas import tpu_sc as plsc

N = 10000
F = 128
E = 320000
HID = 128
OUT = 2
HOP = 42
NC = 2
NS = 16
N_PAD = 10240
DUMMY = N_PAD - 1
CHUNK = 128
E_PAD = 323584
RPT = N_PAD // NS
BLK = 512

_mesh = plsc.VectorSubcoreMesh(
    core_axis_name="c", subcore_axis_name="s", num_cores=NC, num_subcores=NS)


def _zero_acc(znf, acc, rbase):
    pltpu.sync_copy(znf.at[pl.ds(rbase, RPT)], acc.at[pl.ds(rbase, RPT)])


def _pipe_spmm(table, gidx, sidx, acc, ig, isx, rows, sems, ebase, nchunk):
    def load_issue(k, b):
        pltpu.sync_copy(gidx.at[pl.ds(b, CHUNK)], ig[k])
        pltpu.sync_copy(sidx.at[pl.ds(b, CHUNK)], isx[k])
        pltpu.async_copy(table.at[ig[k]], rows[k], sems[k])

    def wait_scatter(k):
        pltpu.make_async_copy(table.at[ig[k]], rows[k], sems[k]).wait()
        pltpu.sync_copy(rows[k], acc.at[isx[k]], add=True)

    load_issue(0, ebase)

    def body(c, carry):
        def step(k):
            @pl.when(c + 1 < nchunk)
            def _():
                load_issue(1 - k, ebase + (c + 1) * CHUNK)
            wait_scatter(k)

        @pl.when(c % 2 == 0)
        def _():
            step(0)

        @pl.when(c % 2 == 1)
        def _():
            step(1)

        return carry

    lax.fori_loop(0, nchunk, body, 0)


def _pipe_hist(sidx, hacc, ones_v, ib, sems, ebase, nchunk):
    def issue(k, b):
        pltpu.async_copy(sidx.at[pl.ds(b, CHUNK)], ib[k], sems[k])

    def wait_scatter(k, b):
        pltpu.make_async_copy(sidx.at[pl.ds(b, CHUNK)], ib[k], sems[k]).wait()
        pltpu.sync_copy(ones_v, hacc.at[ib[k]], add=True)

    issue(0, ebase)

    def body(c, carry):
        b = ebase + c * CHUNK

        def step(k):
            @pl.when(c + 1 < nchunk)
            def _():
                issue(1 - k, b + CHUNK)
            wait_scatter(k, b)

        @pl.when(c % 2 == 0)
        def _():
            step(0)

        @pl.when(c % 2 == 1)
        def _():
            step(1)

        return carry

    lax.fori_loop(0, nchunk, body, 0)


def _sc0_body(srcx, dstx, znf, onef,
              h_out,
              ib0, ib1, ones_v, hacc, semi0, semi1):
    cid = lax.axis_index("c")
    sid = lax.axis_index("s")
    rbase = sid * RPT
    _zero_acc(znf, hacc, rbase)
    pltpu.sync_copy(onef, ones_v)
    plsc.subcore_barrier()
    per_t = E_PAD // NS
    ebase = sid * per_t
    nchunk = per_t // CHUNK

    @pl.when(cid == 0)
    def _():
        _pipe_hist(srcx, hacc, ones_v, (ib0, ib1), (semi0, semi1),
                   ebase, nchunk)

    @pl.when(cid == 1)
    def _():
        _pipe_hist(dstx, hacc, ones_v, (ib0, ib1), (semi0, semi1),
                   ebase, nchunk)

    plsc.subcore_barrier()
    pltpu.sync_copy(hacc.at[pl.ds(rbase, RPT)], h_out.at[cid, pl.ds(rbase, RPT)])


def _sc1_body(table, srcx, dstx, znf,
              s1_out,
              ig0, ig1, is0, is1, r0, r1, acc, sem0, sem1):
    cid = lax.axis_index("c")
    sid = lax.axis_index("s")
    wid = sid * NC + cid
    rbase = sid * RPT
    _zero_acc(znf, acc, rbase)
    plsc.subcore_barrier()
    per_w = E_PAD // (NC * NS)
    _pipe_spmm(table, dstx, srcx, acc, (ig0, ig1), (is0, is1), (r0, r1),
               (sem0, sem1), wid * per_w, per_w // CHUNK)
    plsc.subcore_barrier()
    pltpu.sync_copy(acc.at[pl.ds(rbase, RPT)], s1_out.at[cid, pl.ds(rbase, RPT)])


def _sc2_body(t_a, t_b, srcx, dstx, znf,
              out2,
              ig0, ig1, is0, is1, r0, r1, acc, sem0, sem1):
    cid = lax.axis_index("c")
    sid = lax.axis_index("s")
    rbase = sid * RPT
    _zero_acc(znf, acc, rbase)
    plsc.subcore_barrier()
    per_t = E_PAD // NS
    ebase = sid * per_t
    nchunk = per_t // CHUNK

    @pl.when(cid == 0)
    def _():
        _pipe_spmm(t_a, dstx, srcx, acc, (ig0, ig1), (is0, is1), (r0, r1),
                   (sem0, sem1), ebase, nchunk)

    @pl.when(cid == 1)
    def _():
        _pipe_spmm(t_b, srcx, dstx, acc, (ig0, ig1), (is0, is1), (r0, r1),
                   (sem0, sem1), ebase, nchunk)

    plsc.subcore_barrier()
    pltpu.sync_copy(acc.at[pl.ds(rbase, RPT)], out2.at[cid, pl.ds(rbase, RPT)])


def _sc3_body(table, srcx, dstx, znf,
              out2,
              ig0, ig1, is0, is1, r0, r1, acc, sem0, sem1):
    cid = lax.axis_index("c")
    sid = lax.axis_index("s")
    wid = sid * NC + cid
    rbase = sid * RPT
    _zero_acc(znf, acc, rbase)
    plsc.subcore_barrier()
    per_w = E_PAD // (NC * NS)
    _pipe_spmm(table, srcx, dstx, acc, (ig0, ig1), (is0, is1), (r0, r1),
               (sem0, sem1), wid * per_w, per_w // CHUNK)
    plsc.subcore_barrier()
    pltpu.sync_copy(acc.at[pl.ds(rbase, RPT)], out2.at[cid, pl.ds(rbase, RPT)])


_f32 = jnp.float32

_sc0 = functools.partial(
    pl.kernel, _sc0_body,
    out_type=jax.ShapeDtypeStruct((NC, N_PAD, F), _f32),
    mesh=_mesh,
    scratch_types=[pltpu.VMEM((CHUNK,), jnp.int32),
                   pltpu.VMEM((CHUNK,), jnp.int32),
                   pltpu.VMEM((CHUNK, F), _f32),
                   pltpu.VMEM_SHARED((N_PAD, F), _f32),
                   pltpu.SemaphoreType.DMA,
                   pltpu.SemaphoreType.DMA],
)()

_SPMM_SCRATCH = [pltpu.VMEM((CHUNK,), jnp.int32),
                 pltpu.VMEM((CHUNK,), jnp.int32),
                 pltpu.VMEM((CHUNK,), jnp.int32),
                 pltpu.VMEM((CHUNK,), jnp.int32),
                 pltpu.VMEM((CHUNK, F), _f32),
                 pltpu.VMEM((CHUNK, F), _f32),
                 pltpu.VMEM_SHARED((N_PAD, F), _f32),
                 pltpu.SemaphoreType.DMA,
                 pltpu.SemaphoreType.DMA]

_sc1 = functools.partial(
    pl.kernel, _sc1_body,
    out_type=jax.ShapeDtypeStruct((NC, N_PAD, F), _f32),
    mesh=_mesh,
    scratch_types=_SPMM_SCRATCH,
)()

_sc2 = functools.partial(
    pl.kernel, _sc2_body,
    out_type=jax.ShapeDtypeStruct((NC, N_PAD, F), _f32),
    mesh=_mesh,
    scratch_types=_SPMM_SCRATCH,
)()

_sc3 = functools.partial(
    pl.kernel, _sc3_body,
    out_type=jax.ShapeDtypeStruct((NC, N_PAD, F), _f32),
    mesh=_mesh,
    scratch_types=_SPMM_SCRATCH,
)()


def _tca_body(x_ref, fi_ref, w1_ref, xw_ref, xw1_ref):
    xw = x_ref[...] * jax.nn.sigmoid(fi_ref[...])
    xw_ref[...] = xw
    xw1_ref[...] = jnp.dot(xw, w1_ref[...], preferred_element_type=_f32)


def _tcb_body(s1_ref, h_ref, xw_ref, xw1_ref, gt_ref,
              hhop_ref, p1_ref, aux_ref):
    s1 = s1_ref[0] + s1_ref[1]
    deg_s = h_ref[0][:, 0:1]
    deg_d = h_ref[1][:, 0:1]
    dsinv = 1.0 / jnp.maximum(deg_s, 1.0)
    mean_nb = s1 * dsinv
    xw = xw_ref[...]
    nx = jnp.sqrt(jnp.sum(xw * xw, axis=1, keepdims=True))
    nm = jnp.sqrt(jnp.sum(mean_nb * mean_nb, axis=1, keepdims=True))
    dot = jnp.sum(xw * mean_nb, axis=1, keepdims=True)
    sim = dot / (jnp.maximum(nx, 1e-12) * jnp.maximum(nm, 1e-12))
    sim = jnp.where(deg_s > 0, sim, 1.0)
    delta = jax.nn.sigmoid(deg_s * (1.0 - sim) / 10.0 - 0.5)
    gate = jax.nn.sigmoid((delta - 0.5) * gt_ref[0, 0] * 10.0)
    dinv = lax.rsqrt(deg_d + 1.0)
    dinv2 = dinv * dinv
    hhop_ref[...] = mean_nb
    p1_ref[...] = dinv * xw1_ref[...]
    lane = lax.broadcasted_iota(jnp.int32, s1.shape, 1)
    aux = jnp.where(lane == 0, gate,
                    jnp.where(lane == 1, dinv,
                              jnp.where(lane == 2, dinv2, dsinv)))
    aux_ref[...] = aux


def _tcc_body(c1_ref, xw1_ref, aux_ref, w2_ref, b1_ref, p2_ref, g1_ref):
    dinv = aux_ref[:, 1:2]
    dinv2 = aux_ref[:, 2:3]
    h1 = jax.nn.relu(dinv * c1_ref[0] + dinv2 * xw1_ref[...] + b1_ref[...])
    g1 = jnp.dot(h1, w2_ref[...], preferred_element_type=_f32)
    g1_ref[...] = g1
    p2_ref[...] = dinv * g1


def _tcd_body(c2_ref, g1_ref, hhop_ref, xw_ref, s2_ref, aux_ref,
              wep_ref, wh0_ref, wh1_ref, bhigh_ref, b2_ref,
              bng_ref, bnb_ref, wcp_ref, bcp_ref, out_ref):
    gate = aux_ref[:, 0:1]
    dinv = aux_ref[:, 1:2]
    dinv2 = aux_ref[:, 2:3]
    dsinv = aux_ref[:, 3:4]
    h_low = dinv * (c2_ref[0] + c2_ref[1]) + dinv2 * g1_ref[...] + b2_ref[...]
    hop2 = s2_ref[0] * dsinv
    h_high = jax.nn.relu(
        jnp.dot(xw_ref[...], wep_ref[...], preferred_element_type=_f32)
        + jnp.dot(hhop_ref[...], wh0_ref[...], preferred_element_type=_f32)
        + jnp.dot(hop2, wh1_ref[...], preferred_element_type=_f32)
        + bhigh_ref[...])
    h = (1.0 - gate) * h_low + gate * h_high
    h = (h / jnp.sqrt(1.0 + 1e-5)) * bng_ref[...] + bnb_ref[...]
    out_ref[...] = jnp.dot(h, wcp_ref[...], preferred_element_type=_f32) + bcp_ref[...]


_GRID = (N_PAD // BLK,)


def _row_spec(w=F):
    return pl.BlockSpec((BLK, w), lambda i: (i, 0))


def _full_spec(shape):
    nd = len(shape)
    return pl.BlockSpec(shape, lambda i: (0,) * nd)


def _part_spec(w=F):
    return pl.BlockSpec((1, BLK, w), lambda i: (0, i, 0))


def _part2_spec(w=F):
    return pl.BlockSpec((2, BLK, w), lambda i: (0, i, 0))


def _slot_spec(slot, w=F):
    return pl.BlockSpec((1, BLK, w), lambda i: (slot, i, 0))


_tca = pl.pallas_call(
    _tca_body, grid=_GRID,
    in_specs=[_row_spec(), _full_spec((1, F)), _full_spec((F, HID))],
    out_specs=[_row_spec(), _row_spec()],
    out_shape=[jax.ShapeDtypeStruct((N_PAD, F), _f32),
               jax.ShapeDtypeStruct((N_PAD, HID), _f32)],
)

_tcb = pl.pallas_call(
    _tcb_body, grid=_GRID,
    in_specs=[_part2_spec(), _part2_spec(),
              _row_spec(), _row_spec(), _full_spec((1, 1))],
    out_specs=[_row_spec(), _row_spec(), _row_spec()],
    out_shape=[jax.ShapeDtypeStruct((N_PAD, F), _f32),
               jax.ShapeDtypeStruct((N_PAD, HID), _f32),
               jax.ShapeDtypeStruct((N_PAD, F), _f32)],
)

_tcc = pl.pallas_call(
    _tcc_body, grid=_GRID,
    in_specs=[_slot_spec(1), _row_spec(), _row_spec(),
              _full_spec((HID, HID)), _full_spec((1, HID))],
    out_specs=[_row_spec(), _row_spec()],
    out_shape=[jax.ShapeDtypeStruct((N_PAD, HID), _f32),
               jax.ShapeDtypeStruct((N_PAD, HID), _f32)],
)

_tcd = pl.pallas_call(
    _tcd_body, grid=_GRID,
    in_specs=[_part2_spec(), _row_spec(), _row_spec(), _row_spec(),
              _slot_spec(0), _row_spec(),
              _full_spec((F, HID)), _full_spec((F, HID)), _full_spec((F, HID)),
              _full_spec((1, HID)), _full_spec((1, HID)),
              _full_spec((1, HID)), _full_spec((1, HID)),
              _full_spec((HID, HID)), _full_spec((1, HID))],
    out_specs=_row_spec(),
    out_shape=jax.ShapeDtypeStruct((N_PAD, HID), _f32),
)


def kernel(x, edge_index, fi, w1, b1, w2, b2, we, be, wh0, bh0, wh1, bh1,
           bn_gamma, bn_beta, wc, bc, gate_temp):
    src = edge_index[0]
    dst = edge_index[1]
    src_p = jnp.full((E_PAD,), DUMMY, jnp.int32).at[:E].set(src)
    dst_p = jnp.full((E_PAD,), DUMMY, jnp.int32).at[:E].set(dst)
    x_p = jnp.zeros((N_PAD, F), _f32).at[:N].set(x)
    znf = jnp.zeros((N_PAD, F), _f32)
    onef = jnp.ones((CHUNK, F), _f32)

    wep = jnp.zeros((F, HID), _f32).at[:, :HOP].set(we)
    wh0p = jnp.zeros((F, HID), _f32).at[:, HOP:2 * HOP].set(wh0)
    wh1p = jnp.zeros((F, HID), _f32).at[:, 2 * HOP:].set(wh1)
    bhigh = jnp.concatenate([be, bh0, bh1]).reshape(1, HID)
    wcp = jnp.zeros((HID, HID), _f32).at[:, :OUT].set(wc)
    bcp = jnp.zeros((HID,), _f32).at[:OUT].set(bc).reshape(1, HID)

    hist = _sc0(src_p, dst_p, znf, onef)
    xw, xw1 = _tca(x_p, fi.reshape(1, F), w1)
    s1p = _sc1(xw, src_p, dst_p, znf)
    hhop, p1, aux = _tcb(s1p, hist, xw, xw1,
                         gate_temp.reshape(1, 1).astype(_f32))
    out2 = _sc2(hhop, p1, src_p, dst_p, znf)
    p2, g1 = _tcc(out2, xw1, aux, w2, b1.reshape(1, HID))
    c2p = _sc3(p2, src_p, dst_p, znf)
    final = _tcd(c2p, g1, hhop, xw, out2, aux, wep, wh0p, wh1p, bhigh,
                 b2.reshape(1, HID), bn_gamma.reshape(1, HID),
                 bn_beta.reshape(1, HID), wcp, bcp)
    return final[:N, :OUT]


# speedup vs baseline: 12.2964x; 1.5566x over previous
import functools

import jax
import jax.numpy as jnp
from jax import lax
from jax.experimental import pallas as pl
from jax.experimental.pallas import tpu as pltpu
from jax.experimental.pallas import tpu_sc as plsc

N = 10000
F = 128
E = 320000
HID = 128
OUT = 2
HOP = 42
NC = 2
NS = 16
N_PAD = 10240
DUMMY = N_PAD - 1
CHUNK = 128
SUB = 1
E_PAD = 327680
RPT = N_PAD // NS
BLK = 512

_mesh = plsc.VectorSubcoreMesh(
    core_axis_name="c", subcore_axis_name="s", num_cores=NC, num_subcores=NS)


def _zero_acc(znf, acc, rbase):
    pltpu.sync_copy(znf.at[pl.ds(rbase, RPT)], acc.at[pl.ds(rbase, RPT)])


def _pipe_spmm(table, gidx, sidx, acc, ig, isx, rows, sems, ebase, nchunk):
    def load_issue(k, b):
        for j in range(SUB):
            pltpu.sync_copy(gidx.at[pl.ds(b + j * CHUNK, CHUNK)], ig[k][j])
            pltpu.sync_copy(sidx.at[pl.ds(b + j * CHUNK, CHUNK)], isx[k][j])
        for j in range(SUB):
            pltpu.async_copy(table.at[ig[k][j]], rows[k][j], sems[k])

    def wait_scatter(k):
        for j in range(SUB):
            pltpu.make_async_copy(table.at[ig[k][j]], rows[k][j],
                                  sems[k]).wait()
            pltpu.sync_copy(rows[k][j], acc.at[isx[k][j]], add=True)

    sstep = SUB * CHUNK
    load_issue(0, ebase)

    def body(c, carry):
        def step(k):
            @pl.when(c + 1 < nchunk)
            def _():
                load_issue(1 - k, ebase + (c + 1) * sstep)
            wait_scatter(k)

        @pl.when(c % 2 == 0)
        def _():
            step(0)

        @pl.when(c % 2 == 1)
        def _():
            step(1)

        return carry

    lax.fori_loop(0, nchunk, body, 0)


def _pipe_hist(sidx, hacc, ones_v, ib, sems, ebase, nchunk):
    def issue(k, b):
        pltpu.async_copy(sidx.at[pl.ds(b, CHUNK)], ib[k], sems[k])

    def wait_scatter(k, b):
        pltpu.make_async_copy(sidx.at[pl.ds(b, CHUNK)], ib[k], sems[k]).wait()
        pltpu.sync_copy(ones_v, hacc.at[ib[k]], add=True)

    issue(0, ebase)

    def body(c, carry):
        b = ebase + c * CHUNK

        def step(k):
            @pl.when(c + 1 < nchunk)
            def _():
                issue(1 - k, b + CHUNK)
            wait_scatter(k, b)

        @pl.when(c % 2 == 0)
        def _():
            step(0)

        @pl.when(c % 2 == 1)
        def _():
            step(1)

        return carry

    lax.fori_loop(0, nchunk, body, 0)


def _sc0_body(srcx, dstx, znf, onef,
              h_out,
              ib0, ib1, ones_v, hacc, semi0, semi1):
    cid = lax.axis_index("c")
    sid = lax.axis_index("s")
    rbase = sid * RPT
    _zero_acc(znf, hacc, rbase)
    pltpu.sync_copy(onef, ones_v)
    plsc.subcore_barrier()
    per_t = E_PAD // NS
    ebase = sid * per_t
    nchunk = per_t // CHUNK

    @pl.when(cid == 0)
    def _():
        _pipe_hist(srcx, hacc, ones_v, (ib0, ib1), (semi0, semi1),
                   ebase, nchunk)

    @pl.when(cid == 1)
    def _():
        _pipe_hist(dstx, hacc, ones_v, (ib0, ib1), (semi0, semi1),
                   ebase, nchunk)

    plsc.subcore_barrier()
    pltpu.sync_copy(hacc.at[pl.ds(rbase, RPT)], h_out.at[cid, pl.ds(rbase, RPT)])


def _sc1_body(table, srcx, dstx, znf,
              s1_out,
              ig0, ig1, is0, is1, r0, r1, acc, sem0, sem1):
    cid = lax.axis_index("c")
    sid = lax.axis_index("s")
    wid = sid * NC + cid
    rbase = sid * RPT
    _zero_acc(znf, acc, rbase)
    plsc.subcore_barrier()
    per_w = E_PAD // (NC * NS)
    _pipe_spmm(table, dstx, srcx, acc,
               ((ig0,), (ig1,)), ((is0,), (is1,)),
               ((r0,), (r1,)), (sem0, sem1),
               wid * per_w, per_w // (SUB * CHUNK))
    plsc.subcore_barrier()
    pltpu.sync_copy(acc.at[pl.ds(rbase, RPT)], s1_out.at[cid, pl.ds(rbase, RPT)])


def _sc2_body(t_a, t_b, srcx, dstx, znf,
              out2,
              ig0, ig1, is0, is1, r0, r1, acc, sem0, sem1):
    cid = lax.axis_index("c")
    sid = lax.axis_index("s")
    rbase = sid * RPT
    _zero_acc(znf, acc, rbase)
    plsc.subcore_barrier()
    per_t = E_PAD // NS
    ebase = sid * per_t
    nchunk = per_t // (SUB * CHUNK)
    ig = ((ig0,), (ig1,))
    isx = ((is0,), (is1,))
    rows = ((r0,), (r1,))

    @pl.when(cid == 0)
    def _():
        _pipe_spmm(t_a, dstx, srcx, acc, ig, isx, rows, (sem0, sem1),
                   ebase, nchunk)

    @pl.when(cid == 1)
    def _():
        _pipe_spmm(t_b, srcx, dstx, acc, ig, isx, rows, (sem0, sem1),
                   ebase, nchunk)

    plsc.subcore_barrier()
    pltpu.sync_copy(acc.at[pl.ds(rbase, RPT)], out2.at[cid, pl.ds(rbase, RPT)])


def _sc3_body(table, srcx, dstx, znf,
              out2,
              ig0, ig1, is0, is1, r0, r1, acc, sem0, sem1):
    cid = lax.axis_index("c")
    sid = lax.axis_index("s")
    wid = sid * NC + cid
    rbase = sid * RPT
    _zero_acc(znf, acc, rbase)
    plsc.subcore_barrier()
    per_w = E_PAD // (NC * NS)
    _pipe_spmm(table, srcx, dstx, acc,
               ((ig0,), (ig1,)), ((is0,), (is1,)),
               ((r0,), (r1,)), (sem0, sem1),
               wid * per_w, per_w // (SUB * CHUNK))
    plsc.subcore_barrier()
    pltpu.sync_copy(acc.at[pl.ds(rbase, RPT)], out2.at[cid, pl.ds(rbase, RPT)])


_f32 = jnp.float32

_sc0 = functools.partial(
    pl.kernel, _sc0_body,
    out_type=jax.ShapeDtypeStruct((NC, N_PAD, F), _f32),
    mesh=_mesh,
    scratch_types=[pltpu.VMEM((CHUNK,), jnp.int32),
                   pltpu.VMEM((CHUNK,), jnp.int32),
                   pltpu.VMEM((CHUNK, F), _f32),
                   pltpu.VMEM_SHARED((N_PAD, F), _f32),
                   pltpu.SemaphoreType.DMA,
                   pltpu.SemaphoreType.DMA],
)()

_SPMM_SCRATCH = ([pltpu.VMEM((CHUNK,), jnp.int32)] * 4
                 + [pltpu.VMEM((CHUNK, F), _f32)] * 2
                 + [pltpu.VMEM_SHARED((N_PAD, F), _f32),
                    pltpu.SemaphoreType.DMA,
                    pltpu.SemaphoreType.DMA])

_sc1 = functools.partial(
    pl.kernel, _sc1_body,
    out_type=jax.ShapeDtypeStruct((NC, N_PAD, F), _f32),
    mesh=_mesh,
    scratch_types=_SPMM_SCRATCH,
)()

_sc2 = functools.partial(
    pl.kernel, _sc2_body,
    out_type=jax.ShapeDtypeStruct((NC, N_PAD, F), _f32),
    mesh=_mesh,
    scratch_types=_SPMM_SCRATCH,
)()

_sc3 = functools.partial(
    pl.kernel, _sc3_body,
    out_type=jax.ShapeDtypeStruct((NC, N_PAD, F), _f32),
    mesh=_mesh,
    scratch_types=_SPMM_SCRATCH,
)()


def _tca_body(x_ref, fi_ref, w1_ref, xw_ref, xw1_ref):
    xw = x_ref[...] * jax.nn.sigmoid(fi_ref[...])
    xw_ref[...] = xw
    xw1_ref[...] = jnp.dot(xw, w1_ref[...], preferred_element_type=_f32)


def _tcb_body(s1_ref, h_ref, xw_ref, xw1_ref, gt_ref,
              hhop_ref, p1_ref, aux_ref):
    s1 = s1_ref[0] + s1_ref[1]
    deg_s = h_ref[0][:, 0:1]
    deg_d = h_ref[1][:, 0:1]
    dsinv = 1.0 / jnp.maximum(deg_s, 1.0)
    mean_nb = s1 * dsinv
    xw = xw_ref[...]
    nx = jnp.sqrt(jnp.sum(xw * xw, axis=1, keepdims=True))
    nm = jnp.sqrt(jnp.sum(mean_nb * mean_nb, axis=1, keepdims=True))
    dot = jnp.sum(xw * mean_nb, axis=1, keepdims=True)
    sim = dot / (jnp.maximum(nx, 1e-12) * jnp.maximum(nm, 1e-12))
    sim = jnp.where(deg_s > 0, sim, 1.0)
    delta = jax.nn.sigmoid(deg_s * (1.0 - sim) / 10.0 - 0.5)
    gate = jax.nn.sigmoid((delta - 0.5) * gt_ref[0, 0] * 10.0)
    dinv = lax.rsqrt(deg_d + 1.0)
    dinv2 = dinv * dinv
    hhop_ref[...] = mean_nb
    p1_ref[...] = dinv * xw1_ref[...]
    lane = lax.broadcasted_iota(jnp.int32, s1.shape, 1)
    aux = jnp.where(lane == 0, gate,
                    jnp.where(lane == 1, dinv,
                              jnp.where(lane == 2, dinv2, dsinv)))
    aux_ref[...] = aux


def _tcc_body(c1_ref, xw1_ref, aux_ref, w2_ref, b1_ref, p2_ref, g1_ref):
    dinv = aux_ref[:, 1:2]
    dinv2 = aux_ref[:, 2:3]
    h1 = jax.nn.relu(dinv * c1_ref[0] + dinv2 * xw1_ref[...] + b1_ref[...])
    g1 = jnp.dot(h1, w2_ref[...], preferred_element_type=_f32)
    g1_ref[...] = g1
    p2_ref[...] = dinv * g1


def _tcd_body(c2_ref, g1_ref, hhop_ref, xw_ref, s2_ref, aux_ref,
              wep_ref, wh0_ref, wh1_ref, bhigh_ref, b2_ref,
              bng_ref, bnb_ref, wcp_ref, bcp_ref, out_ref):
    gate = aux_ref[:, 0:1]
    dinv = aux_ref[:, 1:2]
    dinv2 = aux_ref[:, 2:3]
    dsinv = aux_ref[:, 3:4]
    h_low = dinv * (c2_ref[0] + c2_ref[1]) + dinv2 * g1_ref[...] + b2_ref[...]
    hop2 = s2_ref[0] * dsinv
    h_high = jax.nn.relu(
        jnp.dot(xw_ref[...], wep_ref[...], preferred_element_type=_f32)
        + jnp.dot(hhop_ref[...], wh0_ref[...], preferred_element_type=_f32)
        + jnp.dot(hop2, wh1_ref[...], preferred_element_type=_f32)
        + bhigh_ref[...])
    h = (1.0 - gate) * h_low + gate * h_high
    h = (h / jnp.sqrt(1.0 + 1e-5)) * bng_ref[...] + bnb_ref[...]
    out_ref[...] = jnp.dot(h, wcp_ref[...], preferred_element_type=_f32) + bcp_ref[...]


_GRID = (N_PAD // BLK,)


def _row_spec(w=F):
    return pl.BlockSpec((BLK, w), lambda i: (i, 0))


def _full_spec(shape):
    nd = len(shape)
    return pl.BlockSpec(shape, lambda i: (0,) * nd)


def _part_spec(w=F):
    return pl.BlockSpec((1, BLK, w), lambda i: (0, i, 0))


def _part2_spec(w=F):
    return pl.BlockSpec((2, BLK, w), lambda i: (0, i, 0))


def _slot_spec(slot, w=F):
    return pl.BlockSpec((1, BLK, w), lambda i: (slot, i, 0))


_tca = pl.pallas_call(
    _tca_body, grid=_GRID,
    in_specs=[_row_spec(), _full_spec((1, F)), _full_spec((F, HID))],
    out_specs=[_row_spec(), _row_spec()],
    out_shape=[jax.ShapeDtypeStruct((N_PAD, F), _f32),
               jax.ShapeDtypeStruct((N_PAD, HID), _f32)],
)

_tcb = pl.pallas_call(
    _tcb_body, grid=_GRID,
    in_specs=[_part2_spec(), _part2_spec(),
              _row_spec(), _row_spec(), _full_spec((1, 1))],
    out_specs=[_row_spec(), _row_spec(), _row_spec()],
    out_shape=[jax.ShapeDtypeStruct((N_PAD, F), _f32),
               jax.ShapeDtypeStruct((N_PAD, HID), _f32),
               jax.ShapeDtypeStruct((N_PAD, F), _f32)],
)

_tcc = pl.pallas_call(
    _tcc_body, grid=_GRID,
    in_specs=[_slot_spec(1), _row_spec(), _row_spec(),
              _full_spec((HID, HID)), _full_spec((1, HID))],
    out_specs=[_row_spec(), _row_spec()],
    out_shape=[jax.ShapeDtypeStruct((N_PAD, HID), _f32),
               jax.ShapeDtypeStruct((N_PAD, HID), _f32)],
)

_tcd = pl.pallas_call(
    _tcd_body, grid=_GRID,
    in_specs=[_part2_spec(), _row_spec(), _row_spec(), _row_spec(),
              _slot_spec(0), _row_spec(),
              _full_spec((F, HID)), _full_spec((F, HID)), _full_spec((F, HID)),
              _full_spec((1, HID)), _full_spec((1, HID)),
              _full_spec((1, HID)), _full_spec((1, HID)),
              _full_spec((HID, HID)), _full_spec((1, HID))],
    out_specs=_row_spec(),
    out_shape=jax.ShapeDtypeStruct((N_PAD, HID), _f32),
)


def kernel(x, edge_index, fi, w1, b1, w2, b2, we, be, wh0, bh0, wh1, bh1,
           bn_gamma, bn_beta, wc, bc, gate_temp):
    src = edge_index[0]
    dst = edge_index[1]
    pad_idx = N + jnp.arange(E_PAD, dtype=jnp.int32) % (N_PAD - N)
    src_p = pad_idx.at[:E].set(src)
    dst_p = pad_idx.at[:E].set(dst)
    x_p = jnp.zeros((N_PAD, F), _f32).at[:N].set(x)
    znf = jnp.zeros((N_PAD, F), _f32)
    onef = jnp.ones((CHUNK, F), _f32)

    wep = jnp.zeros((F, HID), _f32).at[:, :HOP].set(we)
    wh0p = jnp.zeros((F, HID), _f32).at[:, HOP:2 * HOP].set(wh0)
    wh1p = jnp.zeros((F, HID), _f32).at[:, 2 * HOP:].set(wh1)
    bhigh = jnp.concatenate([be, bh0, bh1]).reshape(1, HID)
    wcp = jnp.zeros((HID, HID), _f32).at[:, :OUT].set(wc)
    bcp = jnp.zeros((HID,), _f32).at[:OUT].set(bc).reshape(1, HID)

    hist = _sc0(src_p, dst_p, znf, onef)
    xw, xw1 = _tca(x_p, fi.reshape(1, F), w1)
    s1p = _sc1(xw, src_p, dst_p, znf)
    hhop, p1, aux = _tcb(s1p, hist, xw, xw1,
                         gate_temp.reshape(1, 1).astype(_f32))
    out2 = _sc2(hhop, p1, src_p, dst_p, znf)
    p2, g1 = _tcc(out2, xw1, aux, w2, b1.reshape(1, HID))
    c2p = _sc3(p2, src_p, dst_p, znf)
    final = _tcd(c2p, g1, hhop, xw, out2, aux, wep, wh0p, wh1p, bhigh,
                 b2.reshape(1, HID), bn_gamma.reshape(1, HID),
                 bn_beta.reshape(1, HID), wcp, bcp)
    return final[:N, :OUT]
